# baseline (speedup 1.0000x reference)
import functools

import jax
import jax.numpy as jnp
from jax import lax
from jax.experimental import pallas as pl
from jax.experimental.pallas import tpu as pltpu
from jax.experimental.pallas import tpu_sc as plsc

N_NODES = 10000
N_EDGES = 160000
D = 256
DH = D // 2

NUM_TILES = 16
CHUNK = 128
EDGES_PER_TILE = 10240
NCHUNK = EDGES_PER_TILE // CHUNK
NCHUNK_P = NCHUNK + 3
E_PAD = NUM_TILES * EDGES_PER_TILE
N_PAD = 10112
NR = N_PAD // 2
NR_PAD = 5120

_mesh = plsc.VectorSubcoreMesh(
    core_axis_name="c", subcore_axis_name="s", num_cores=2, num_subcores=NUM_TILES
)


_SC_OUT_TYPE = [
        jax.ShapeDtypeStruct((NR_PAD, DH), jnp.float32),
        jax.ShapeDtypeStruct((NR_PAD, DH), jnp.float32),
        jax.ShapeDtypeStruct((NR_PAD, DH), jnp.float32),
        jax.ShapeDtypeStruct((NR_PAD, DH), jnp.float32),
        jax.ShapeDtypeStruct((CHUNK, CHUNK), jnp.float32),
        jax.ShapeDtypeStruct((CHUNK, CHUNK), jnp.float32),
]
_SC_SCRATCH = (
        [pltpu.VMEM((1, CHUNK), jnp.int32)] * 6
        + [pltpu.VMEM((CHUNK, DH), jnp.float32)] * 3
        + [
            pltpu.VMEM((CHUNK, CHUNK), jnp.float32),
            pltpu.VMEM((1, CHUNK), jnp.int32),
            pltpu.VMEM_SHARED((NR_PAD, DH), jnp.float32),
            pltpu.VMEM_SHARED((CHUNK, CHUNK), jnp.float32),
        ]
        + [pltpu.SemaphoreType.DMA] * 6
)


def _sc_agg_body(fts_lo, fts_hi, u4, v3, z_agg, iota_in,
            a_l0, a_l1, a_h0, a_h1, cnt0, cnt1,
            u_0, v_0, u_1, v_1, u_2, v_2, buf_0, buf_1, buf_2,
            hist, iota_v, agg_s, cnt_g,
            sem_g0, sem_g1, sem_g2, sem_i0, sem_i1, sem_i2):
    c = lax.axis_index("c")
    sid = lax.axis_index("s")
    zrows = NR_PAD // NUM_TILES
    crows = CHUNK // NUM_TILES

    pltpu.sync_copy(iota_in, iota_v)
    ones16 = jnp.ones((16,), jnp.float32)

    def zero_hist(i, carry):
        for k in range(CHUNK // 16):
            hist[i, pl.ds(k * 16, 16)] = jnp.zeros((16,), jnp.float32)
        return carry

    lax.fori_loop(0, CHUNK, zero_hist, 0)
    pltpu.sync_copy(z_agg.at[pl.ds(sid * crows, crows)],
                    cnt_g.at[pl.ds(sid * crows, crows), pl.ds(0, CHUNK)])

    def one_pass(r, fts_h, agg_out, cnt_out):
        pltpu.sync_copy(z_agg.at[pl.ds(sid * zrows, zrows)],
                        agg_s.at[pl.ds(sid * zrows, zrows)])
        plsc.subcore_barrier()
        do_hist = cnt_out is not None

        S = ((u_0, v_0, buf_0, sem_g0, sem_i0),
             (u_1, v_1, buf_1, sem_g1, sem_i1),
             (u_2, v_2, buf_2, sem_g2, sem_i2))

        def idx_start(g, s):
            pltpu.async_copy(u4.at[r, sid, pl.ds(g, 1)], s[0], s[4])
            pltpu.async_copy(v3.at[sid, pl.ds(g, 1)], s[1], s[4])

        def idx_wait(s):
            pltpu.make_async_copy(u4.at[r, sid, pl.ds(0, 1)], s[0], s[4]).wait()
            pltpu.make_async_copy(v3.at[sid, pl.ds(0, 1)], s[1], s[4]).wait()

        def gather_start(s):
            pltpu.async_copy(fts_h.at[s[1].at[0]], s[2], s[3])

        def gather_wait(s):
            pltpu.make_async_copy(fts_h.at[s[1].at[0]], s[2], s[3]).wait()

        def half(g, o):
            s_c = S[o]
            s_n = S[(o + 2) % 3]
            gather_wait(s_c)
            idx_wait(s_n)
            gather_start(s_n)
            pltpu.sync_copy(s_c[2], agg_s.at[s_c[0].at[0]], add=True)
            if do_hist:
                for k in range(CHUNK // 16):
                    u16 = s_c[0][0, pl.ds(k * 16, 16)]
                    rows = lax.shift_right_logical(u16, 7)
                    cols = lax.bitwise_and(u16, 127)
                    plsc.addupdate_scatter(hist, (rows, cols), ones16)
            idx_start(g + 3, s_c)

        pltpu.sync_copy(u4.at[r, sid, pl.ds(0, 1)], u_0)
        pltpu.sync_copy(v3.at[sid, pl.ds(0, 1)], v_0)
        pltpu.sync_copy(u4.at[r, sid, pl.ds(1, 1)], u_1)
        pltpu.sync_copy(v3.at[sid, pl.ds(1, 1)], v_1)
        idx_start(2, S[2])
        gather_start(S[0])
        gather_start(S[1])
        half(0, 0)
        half(1, 1)

        def triple(p, carry):
            g = 3 * p + 2
            half(g, 2)
            half(g + 1, 0)
            half(g + 2, 1)
            return carry

        lax.fori_loop(0, (NCHUNK - 2) // 3, triple, 0)
        idx_wait(S[1])
        gather_wait(S[2])
        gather_wait(S[0])
        if do_hist:
            pltpu.sync_copy(hist, cnt_g.at[iota_v.at[0]], add=True)
        plsc.subcore_barrier()
        pltpu.sync_copy(agg_s.at[pl.ds(sid * zrows, zrows)],
                        agg_out.at[pl.ds(sid * zrows, zrows)])
        if do_hist:
            pltpu.sync_copy(cnt_g.at[pl.ds(sid * crows, crows)],
                            cnt_out.at[pl.ds(sid * crows, crows)])

    @pl.when(c == 0)
    def _():
        one_pass(0, fts_lo, a_l0, cnt0)
        one_pass(1, fts_lo, a_l1, None)

    @pl.when(c == 1)
    def _():
        one_pass(0, fts_hi, a_h0, None)
        one_pass(1, fts_hi, a_h1, cnt1)


_sc_agg = pl.kernel(_sc_agg_body, out_type=_SC_OUT_TYPE, mesh=_mesh,
                    scratch_types=_SC_SCRATCH,
                    compiler_params=pltpu.CompilerParams(
                        needs_layout_passes=False))


BN = 1264


def _dense(fts, agg_lo, agg_hi, cnt, wlt, wr_lo, wr_hi, out_ref):
    inv = 1.0 / jnp.maximum(cnt[:, 0:1], 1.0)
    h = jnp.dot(fts[...], wlt[...], preferred_element_type=jnp.float32)
    agg_h = (jnp.dot(agg_lo[...], wr_lo[...], preferred_element_type=jnp.float32)
             + jnp.dot(agg_hi[...], wr_hi[...], preferred_element_type=jnp.float32))
    h = h + agg_h * inv
    nrm = lax.rsqrt(jnp.sum(h * h, axis=1, keepdims=True))
    out_ref[...] = h * nrm


_dense_call = pl.pallas_call(
    _dense,
    grid=(N_PAD // BN,),
    in_specs=[
        pl.BlockSpec((BN, D), lambda i: (i, 0)),
        pl.BlockSpec((BN, DH), lambda i: (i, 0)),
        pl.BlockSpec((BN, DH), lambda i: (i, 0)),
        pl.BlockSpec((BN, 1), lambda i: (i, 0)),
        pl.BlockSpec((D, D), lambda i: (0, 0)),
        pl.BlockSpec((DH, D), lambda i: (0, 0)),
        pl.BlockSpec((DH, D), lambda i: (0, 0)),
    ],
    out_specs=pl.BlockSpec((BN, D), lambda i: (i, 0)),
    out_shape=jax.ShapeDtypeStruct((N_PAD, D), jnp.float32),
)


def kernel(fts, edge_index, W_l, W_r):
    ei = edge_index.astype(jnp.int32)
    u = ei[0]
    v = ei[1]
    npad = E_PAD - N_EDGES
    u_pad = jnp.concatenate([u, jnp.full((npad,), N_NODES, jnp.int32)])
    v_pad = jnp.concatenate([v, jnp.zeros((npad,), jnp.int32)])
    dummy = jnp.int32(NR_PAD - 1)
    u_r0 = jnp.where(u_pad < NR, u_pad, dummy)
    u_r1_raw = u_pad - NR
    u_r1 = jnp.where((u_r1_raw >= 0) & (u_r1_raw < NR_PAD - 1), u_r1_raw, dummy)
    u4 = jnp.stack([u_r0, u_r1]).reshape(2, NUM_TILES, NCHUNK, CHUNK)
    u4 = jnp.pad(u4, ((0, 0), (0, 0), (0, NCHUNK_P - NCHUNK), (0, 0)))
    v3 = v_pad.reshape(NUM_TILES, NCHUNK, CHUNK)
    v3 = jnp.pad(v3, ((0, 0), (0, NCHUNK_P - NCHUNK), (0, 0)))
    fts_p = jnp.concatenate([fts, jnp.zeros((N_PAD - N_NODES, D), jnp.float32)])
    fts_lo = fts_p[:, :DH]
    fts_hi = fts_p[:, DH:]
    z_agg = jnp.zeros((NR_PAD, DH), jnp.float32)
    iota_in = jnp.arange(CHUNK, dtype=jnp.int32).reshape(1, CHUNK)

    a_l0, a_l1, a_h0, a_h1, c0, c1 = _sc_agg(fts_lo, fts_hi, u4, v3, z_agg,
                                             iota_in)

    agg_lo = jnp.concatenate([a_l0[:NR], a_l1[:NR]])
    agg_hi = jnp.concatenate([a_h0[:NR], a_h1[:NR]])
    cnt = jnp.concatenate([c0.reshape(-1)[:NR], c1.reshape(-1)[:NR]])[:, None]

    wlt = W_l.T
    wrt = W_r.T
    out = _dense_call(fts_p, agg_lo, agg_hi, cnt, wlt, wrt[:DH], wrt[DH:])
    return out[:N_NODES]


# speedup vs baseline: 1.2025x; 1.2025x over previous
import functools

import jax
import jax.numpy as jnp
from jax import lax
from jax.experimental import pallas as pl
from jax.experimental.pallas import tpu as pltpu
from jax.experimental.pallas import tpu_sc as plsc

N_NODES = 10000
N_EDGES = 160000
D = 256
DH = D // 2

NUM_TILES = 16
CHUNK = 128
EDGES_PER_TILE = 10240
NCHUNK = EDGES_PER_TILE // CHUNK
NCHUNK_P = NCHUNK + 2
E_PAD = NUM_TILES * EDGES_PER_TILE
N_PAD = 10112
NR = N_PAD // 2
NR_PAD = 5120

_mesh = plsc.VectorSubcoreMesh(
    core_axis_name="c", subcore_axis_name="s", num_cores=2, num_subcores=NUM_TILES
)


_SC_OUT_TYPE = [
        jax.ShapeDtypeStruct((NR_PAD, DH), jnp.float32),
        jax.ShapeDtypeStruct((NR_PAD, DH), jnp.float32),
        jax.ShapeDtypeStruct((NR_PAD, DH), jnp.float32),
        jax.ShapeDtypeStruct((NR_PAD, DH), jnp.float32),
        jax.ShapeDtypeStruct((CHUNK, CHUNK), jnp.float32),
        jax.ShapeDtypeStruct((CHUNK, CHUNK), jnp.float32),
]
_SC_SCRATCH = [
        pltpu.VMEM((1, CHUNK), jnp.int32),
        pltpu.VMEM((1, CHUNK), jnp.int32),
        pltpu.VMEM((1, CHUNK), jnp.int32),
        pltpu.VMEM((1, CHUNK), jnp.int32),
        pltpu.VMEM((CHUNK, DH), jnp.float32),
        pltpu.VMEM((CHUNK, DH), jnp.float32),
        pltpu.VMEM((CHUNK, CHUNK), jnp.float32),
        pltpu.VMEM((1, CHUNK), jnp.int32),
        pltpu.VMEM_SHARED((NR_PAD, DH), jnp.float32),
        pltpu.VMEM_SHARED((CHUNK, CHUNK), jnp.float32),
        pltpu.SemaphoreType.DMA,
        pltpu.SemaphoreType.DMA,
        pltpu.SemaphoreType.DMA,
        pltpu.SemaphoreType.DMA,
]


def _sc_agg_body(fts_lo, fts_hi, u4, v3, z_agg, iota_in,
            a_l0, a_l1, a_h0, a_h1, cnt0, cnt1,
            u_a, v_a, u_b, v_b, buf_a, buf_b, hist, iota_v, agg_s, cnt_g,
            sem_a, sem_b, sem_ia, sem_ib):
    c = lax.axis_index("c")
    sid = lax.axis_index("s")
    zrows = NR_PAD // NUM_TILES
    crows = CHUNK // NUM_TILES

    pltpu.sync_copy(iota_in, iota_v)
    ones16 = jnp.ones((16,), jnp.float32)

    def zero_hist(i, carry):
        for k in range(CHUNK // 16):
            hist[i, pl.ds(k * 16, 16)] = jnp.zeros((16,), jnp.float32)
        return carry

    lax.fori_loop(0, CHUNK, zero_hist, 0)
    pltpu.sync_copy(z_agg.at[pl.ds(sid * crows, crows)],
                    cnt_g.at[pl.ds(sid * crows, crows), pl.ds(0, CHUNK)])

    def one_pass(r, fts_h, agg_out, cnt_out):
        pltpu.sync_copy(z_agg.at[pl.ds(sid * zrows, zrows)],
                        agg_s.at[pl.ds(sid * zrows, zrows)])
        plsc.subcore_barrier()
        do_hist = cnt_out is not None

        def idx_start(g, u_d, v_d, sem_i):
            pltpu.async_copy(u4.at[r, sid, pl.ds(g, 1)], u_d, sem_i)
            pltpu.async_copy(v3.at[sid, pl.ds(g, 1)], v_d, sem_i)

        def idx_wait(u_d, v_d, sem_i):
            pltpu.make_async_copy(u4.at[r, sid, pl.ds(0, 1)], u_d, sem_i).wait()
            pltpu.make_async_copy(v3.at[sid, pl.ds(0, 1)], v_d, sem_i).wait()

        def gather_wait(v_d, buf, sem_g):
            pltpu.make_async_copy(fts_h.at[v_d.at[0]], buf, sem_g).wait()

        def half(g_next, u_c, v_c, buf_c, sem_c, u_n, v_n, buf_n, sem_n,
                 sem_in, sem_ic):
            idx_wait(u_n, v_n, sem_in)
            gather_wait(v_c, buf_c, sem_c)
            pltpu.async_copy(fts_h.at[v_n.at[0]], buf_n, sem_n)
            pltpu.sync_copy(buf_c, agg_s.at[u_c.at[0]], add=True)
            if do_hist:
                for k in range(CHUNK // 16):
                    u16 = u_c[0, pl.ds(k * 16, 16)]
                    rows = lax.shift_right_logical(u16, 7)
                    cols = lax.bitwise_and(u16, 127)
                    plsc.addupdate_scatter(hist, (rows, cols), ones16)
            idx_start(g_next + 1, u_c, v_c, sem_ic)

        pltpu.sync_copy(u4.at[r, sid, pl.ds(0, 1)], u_a)
        pltpu.sync_copy(v3.at[sid, pl.ds(0, 1)], v_a)
        pltpu.async_copy(fts_h.at[v_a.at[0]], buf_a, sem_a)
        idx_start(1, u_b, v_b, sem_ib)

        def pair(p, carry):
            g = 2 * p
            half(g + 1, u_a, v_a, buf_a, sem_a, u_b, v_b, buf_b, sem_b,
                 sem_ib, sem_ia)
            half(g + 2, u_b, v_b, buf_b, sem_b, u_a, v_a, buf_a, sem_a,
                 sem_ia, sem_ib)
            return carry

        lax.fori_loop(0, NCHUNK // 2, pair, 0)
        idx_wait(u_b, v_b, sem_ib)
        gather_wait(v_a, buf_a, sem_a)
        if do_hist:
            pltpu.sync_copy(hist, cnt_g.at[iota_v.at[0]], add=True)
        plsc.subcore_barrier()
        pltpu.sync_copy(agg_s.at[pl.ds(sid * zrows, zrows)],
                        agg_out.at[pl.ds(sid * zrows, zrows)])
        if do_hist:
            pltpu.sync_copy(cnt_g.at[pl.ds(sid * crows, crows)],
                            cnt_out.at[pl.ds(sid * crows, crows)])

    @pl.when(c == 0)
    def _():
        one_pass(0, fts_lo, a_l0, cnt0)
        one_pass(1, fts_lo, a_l1, None)

    @pl.when(c == 1)
    def _():
        one_pass(0, fts_hi, a_h0, None)
        one_pass(1, fts_hi, a_h1, cnt1)


_sc_agg = pl.kernel(_sc_agg_body, out_type=_SC_OUT_TYPE, mesh=_mesh,
                    scratch_types=_SC_SCRATCH,
                    compiler_params=pltpu.CompilerParams(
                        needs_layout_passes=False))


BN = 1264


def _dense(fts, agg_lo, agg_hi, cnt, wlt, wr_lo, wr_hi, out_ref):
    inv = 1.0 / jnp.maximum(cnt[:, 0:1], 1.0)
    h = jnp.dot(fts[...], wlt[...], preferred_element_type=jnp.float32)
    agg_h = (jnp.dot(agg_lo[...], wr_lo[...], preferred_element_type=jnp.float32)
             + jnp.dot(agg_hi[...], wr_hi[...], preferred_element_type=jnp.float32))
    h = h + agg_h * inv
    nrm = lax.rsqrt(jnp.sum(h * h, axis=1, keepdims=True))
    out_ref[...] = h * nrm


_dense_call = pl.pallas_call(
    _dense,
    grid=(N_PAD // BN,),
    in_specs=[
        pl.BlockSpec((BN, D), lambda i: (i, 0)),
        pl.BlockSpec((BN, DH), lambda i: (i, 0)),
        pl.BlockSpec((BN, DH), lambda i: (i, 0)),
        pl.BlockSpec((BN, 1), lambda i: (i, 0)),
        pl.BlockSpec((D, D), lambda i: (0, 0)),
        pl.BlockSpec((DH, D), lambda i: (0, 0)),
        pl.BlockSpec((DH, D), lambda i: (0, 0)),
    ],
    out_specs=pl.BlockSpec((BN, D), lambda i: (i, 0)),
    out_shape=jax.ShapeDtypeStruct((N_PAD, D), jnp.float32),
)


def kernel(fts, edge_index, W_l, W_r):
    ei = edge_index.astype(jnp.int32)
    u = ei[0]
    v = ei[1]
    npad = E_PAD - N_EDGES
    u_pad = jnp.concatenate([u, jnp.full((npad,), N_NODES, jnp.int32)])
    v_pad = jnp.concatenate([v, jnp.zeros((npad,), jnp.int32)])
    dummy = jnp.int32(NR_PAD - 1)
    u_r0 = jnp.where(u_pad < NR, u_pad, dummy)
    u_r1_raw = u_pad - NR
    u_r1 = jnp.where((u_r1_raw >= 0) & (u_r1_raw < NR_PAD - 1), u_r1_raw, dummy)
    u4 = jnp.stack([u_r0, u_r1]).reshape(2, NUM_TILES, NCHUNK, CHUNK)
    u4 = jnp.pad(u4, ((0, 0), (0, 0), (0, NCHUNK_P - NCHUNK), (0, 0)))
    v3 = v_pad.reshape(NUM_TILES, NCHUNK, CHUNK)
    v3 = jnp.pad(v3, ((0, 0), (0, NCHUNK_P - NCHUNK), (0, 0)))
    fts_p = jnp.concatenate([fts, jnp.zeros((N_PAD - N_NODES, D), jnp.float32)])
    fts_lo = fts_p[:, :DH]
    fts_hi = fts_p[:, DH:]
    z_agg = jnp.zeros((NR_PAD, DH), jnp.float32)
    iota_in = jnp.arange(CHUNK, dtype=jnp.int32).reshape(1, CHUNK)

    a_l0, a_l1, a_h0, a_h1, c0, c1 = _sc_agg(fts_lo, fts_hi, u4, v3, z_agg,
                                             iota_in)

    agg_lo = jnp.concatenate([a_l0[:NR], a_l1[:NR]])
    agg_hi = jnp.concatenate([a_h0[:NR], a_h1[:NR]])
    cnt = jnp.concatenate([c0.reshape(-1)[:NR], c1.reshape(-1)[:NR]])[:, None]

    wlt = W_l.T
    wrt = W_r.T
    out = _dense_call(fts_p, agg_lo, agg_hi, cnt, wlt, wrt[:DH], wrt[DH:])
    return out[:N_NODES]
